# Initial kernel scaffold; baseline (speedup 1.0000x reference)
import jax
import jax.numpy as jnp
from jax.experimental import pallas as pl


def kernel(encoded_row, encoded_col, W, robot_lot_idx, robot_lot_step, flow, num_lot_type, num_step):
    raise NotImplementedError("write your pallas kernel here")


# speedup vs baseline: 2.4287x; 2.4287x over previous
import functools

import jax
import jax.numpy as jnp
from jax import lax
from jax.experimental import pallas as pl
from jax.experimental.pallas import tpu as pltpu
from jax.experimental.pallas import tpu_sc as plsc

_NC = 2
_NS = 16
_NW = _NC * _NS
_L = 16


def _sc_gather(row_flat, col_flat, flow_flat, idx_flat, step_flat, nlt_vec, nst_vec):
    P = idx_flat.shape[0]
    D = row_flat.shape[1]
    C = col_flat.shape[0] // (P // 2)
    PP = P // _NW
    H = PP // 2
    NCH = PP // _L

    mesh = plsc.VectorSubcoreMesh(core_axis_name="c", subcore_axis_name="s")

    @functools.partial(
        pl.kernel,
        mesh=mesh,
        out_type=(
            jax.ShapeDtypeStruct((P, D), jnp.float32),
            jax.ShapeDtypeStruct((P, D), jnp.float32),
            jax.ShapeDtypeStruct((P,), jnp.float32),
            jax.ShapeDtypeStruct((P,), jnp.float32),
        ),
        scratch_types=[
            pltpu.VMEM((PP,), jnp.int32),
            pltpu.VMEM((PP,), jnp.int32),
            pltpu.VMEM((16,), jnp.int32),
            pltpu.VMEM((16,), jnp.int32),
            pltpu.VMEM((H,), jnp.int32),
            pltpu.VMEM((H,), jnp.int32),
            pltpu.VMEM((H,), jnp.int32),
            pltpu.VMEM((H,), jnp.int32),
            pltpu.VMEM((H,), jnp.int32),
            pltpu.VMEM((H,), jnp.int32),
            pltpu.VMEM((PP,), jnp.float32),
            pltpu.VMEM((PP,), jnp.float32),
            pltpu.VMEM((PP,), jnp.int32),
            pltpu.VMEM((PP,), jnp.int32),
            pltpu.VMEM((PP, D), jnp.float32),
            pltpu.VMEM((PP, D), jnp.float32),
            pltpu.SemaphoreType.DMA,
            pltpu.SemaphoreType.DMA,
            pltpu.SemaphoreType.DMA,
        ],
    )
    def k(row_hbm, col_hbm, flow_hbm, idx_hbm, step_hbm, nlt_hbm, nst_hbm,
          rows_out, cols_out, mrow_out, mcol_out,
          idx_v, step_v, nlt_v, nst_v, ridx_a, ridx_b, fidx_a, fidx_b,
          cidx_a, cidx_b, mrow_v, mcol_v, dummy_v, ns_v, rows_v, cols_v,
          sem_rows, sem_flow, sem_cols):
        wid = lax.axis_index("s") * _NC + lax.axis_index("c")
        base = wid * PP
        pltpu.sync_copy(idx_hbm.at[pl.ds(base, PP)], idx_v)
        pltpu.sync_copy(step_hbm.at[pl.ds(base, PP)], step_v)
        pltpu.sync_copy(nlt_hbm, nlt_v)
        pltpu.sync_copy(nst_hbm, nst_v)
        nlt = nlt_v[...]
        nst = nst_v[...]
        iota = lax.broadcasted_iota(jnp.int32, (16,), 0)

        for c in range(NCH):
            sl = pl.ds(c * _L, _L)
            iv = idx_v[sl]
            sv = step_v[sl]
            p = (base + c * _L) + iota
            b = p >> 1
            valid = lax.shift_right_logical(iv - nlt - 1, 31)
            lif = iv * valid
            ridx = (b << 6) + lif
            tgt = ridx_a if c < NCH // 2 else ridx_b
            tgt[pl.ds((c % (NCH // 2)) * _L, _L)] = ridx
            mrow_v[sl] = valid.astype(jnp.float32)
            nsp = sv + 1
            dummy = lax.shift_right_logical(nst - nsp, 31)
            dns = nsp * (1 - dummy)
            fidx = (b << 11) + (lif << 5) + dns
            ftgt = fidx_a if c < NCH // 2 else fidx_b
            ftgt[pl.ds((c % (NCH // 2)) * _L, _L)] = fidx
            dummy_v[sl] = dummy

        cp_fa = pltpu.async_copy(flow_hbm.at[fidx_a], ns_v.at[pl.ds(0, H)], sem_flow)
        cp_fb = pltpu.async_copy(flow_hbm.at[fidx_b], ns_v.at[pl.ds(H, H)], sem_flow)
        cp_ra = pltpu.async_copy(row_hbm.at[ridx_a], rows_v.at[pl.ds(0, H)], sem_rows)
        cp_rb = pltpu.async_copy(row_hbm.at[ridx_b], rows_v.at[pl.ds(H, H)], sem_rows)
        cp_fa.wait()
        cp_fb.wait()

        for c in range(NCH):
            sl = pl.ds(c * _L, _L)
            ns = ns_v[sl]
            dummy = dummy_v[sl]
            ns_pos = 1 - lax.shift_right_logical(ns - 1, 31)
            ns_inb = lax.shift_right_logical(ns - C - 1, 31)
            validc = (1 - dummy) * ns_pos * ns_inb
            p = (base + c * _L) + iota
            cidx = ((p >> 1) << 6) + (ns - 1) * validc
            tgt = cidx_a if c < NCH // 2 else cidx_b
            tgt[pl.ds((c % (NCH // 2)) * _L, _L)] = cidx
            mcol_v[sl] = validc.astype(jnp.float32)

        cp_ca = pltpu.async_copy(col_hbm.at[cidx_a], cols_v.at[pl.ds(0, H)], sem_cols)
        cp_cb = pltpu.async_copy(col_hbm.at[cidx_b], cols_v.at[pl.ds(H, H)], sem_cols)

        pltpu.sync_copy(mrow_v, mrow_out.at[pl.ds(base, PP)])
        pltpu.sync_copy(mcol_v, mcol_out.at[pl.ds(base, PP)])
        cp_ra.wait()
        cp_rb.wait()
        pltpu.sync_copy(rows_v, rows_out.at[pl.ds(base, PP)])
        cp_ca.wait()
        cp_cb.wait()
        pltpu.sync_copy(cols_v, cols_out.at[pl.ds(base, PP)])

    return k(row_flat, col_flat, flow_flat, idx_flat, step_flat, nlt_vec, nst_vec)


def _tc_combine_matmul(rows2, cols2, mrow2, mcol2, W):
    B, TD = rows2.shape
    D = W.shape[0]
    BM = 512

    def body(r_ref, c_ref, mr_ref, mc_ref, w_ref, o_ref):
        mr = mr_ref[...]
        mc = mc_ref[...]
        mre = jnp.concatenate(
            [jnp.broadcast_to(mr[:, 0:1], (BM, D)),
             jnp.broadcast_to(mr[:, 1:2], (BM, D))], axis=1)
        mce = jnp.concatenate(
            [jnp.broadcast_to(mc[:, 0:1], (BM, D)),
             jnp.broadcast_to(mc[:, 1:2], (BM, D))], axis=1)
        e = r_ref[...] * mre + c_ref[...] * mce
        o_ref[...] = lax.dot_general(
            e, w_ref[...], (((1,), (1,)), ((), ())),
            preferred_element_type=jnp.float32)

    return pl.pallas_call(
        body,
        grid=(B // BM,),
        in_specs=[
            pl.BlockSpec((BM, TD), lambda i: (i, 0)),
            pl.BlockSpec((BM, TD), lambda i: (i, 0)),
            pl.BlockSpec((BM, 2), lambda i: (i, 0)),
            pl.BlockSpec((BM, 2), lambda i: (i, 0)),
            pl.BlockSpec((D, TD), lambda i: (0, 0)),
        ],
        out_specs=pl.BlockSpec((BM, D), lambda i: (i, 0)),
        out_shape=jax.ShapeDtypeStruct((B, D), jnp.float32),
    )(rows2, cols2, mrow2, mcol2, W)


def kernel(encoded_row, encoded_col, W, robot_lot_idx, robot_lot_step, flow,
           num_lot_type, num_step):
    B, R, D = encoded_row.shape
    C = encoded_col.shape[1]
    NL, NSTP = flow.shape[1], flow.shape[2]

    row_flat = encoded_row.reshape(B * R, D)
    col_flat = encoded_col.reshape(B * C, D)
    flow_flat = flow.reshape(B * NL * NSTP).astype(jnp.int32)
    idx_flat = robot_lot_idx.reshape(-1).astype(jnp.int32)
    step_flat = robot_lot_step.reshape(-1).astype(jnp.int32)
    nlt_vec = jnp.full((16,), num_lot_type, jnp.int32)
    nst_vec = jnp.full((16,), num_step, jnp.int32)

    rows_g, cols_g, mrow, mcol = _sc_gather(
        row_flat, col_flat, flow_flat, idx_flat, step_flat, nlt_vec, nst_vec)

    rows2 = rows_g.reshape(B, 2 * D)
    cols2 = cols_g.reshape(B, 2 * D)
    mrow2 = mrow.reshape(B, 2)
    mcol2 = mcol.reshape(B, 2)
    return _tc_combine_matmul(rows2, cols2, mrow2, mcol2, W)


# speedup vs baseline: 2.6112x; 1.0751x over previous
import functools

import jax
import jax.numpy as jnp
from jax import lax
from jax.experimental import pallas as pl
from jax.experimental.pallas import tpu as pltpu
from jax.experimental.pallas import tpu_sc as plsc

_NC = 2
_NS = 16
_NW = _NC * _NS
_L = 16


def _sc_gather(row_flat, col_flat, flow16, idx0, idx1, step0, step1,
               nlt_vec, nst_vec):
    B = idx0.shape[0]
    D = row_flat.shape[1]
    C = col_flat.shape[0] // B
    NB = B // _NW
    PP = 2 * NB
    NCH = PP // _L
    ACH = NB // _L

    mesh = plsc.VectorSubcoreMesh(core_axis_name="c", subcore_axis_name="s")

    gd = lax.GatherDimensionNumbers(
        offset_dims=(), collapsed_slice_dims=(0,), start_index_map=(0,))

    def take16(vec, lane_idx):
        return lax.gather(vec, lane_idx[:, None], gd, (1,),
                          mode=lax.GatherScatterMode.PROMISE_IN_BOUNDS)

    @functools.partial(
        pl.kernel,
        mesh=mesh,
        out_type=(
            jax.ShapeDtypeStruct((B, D), jnp.float32),
            jax.ShapeDtypeStruct((B, D), jnp.float32),
            jax.ShapeDtypeStruct((B, D), jnp.float32),
            jax.ShapeDtypeStruct((B, D), jnp.float32),
            jax.ShapeDtypeStruct((B,), jnp.int32),
            jax.ShapeDtypeStruct((B,), jnp.int32),
        ),
        scratch_types=[
            pltpu.VMEM((PP,), jnp.int32),
            pltpu.VMEM((PP,), jnp.int32),
            pltpu.VMEM((16,), jnp.int32),
            pltpu.VMEM((16,), jnp.int32),
            pltpu.VMEM((NB,), jnp.int32),
            pltpu.VMEM((NB,), jnp.int32),
            pltpu.VMEM((NB,), jnp.int32),
            pltpu.VMEM((NB,), jnp.int32),
            pltpu.VMEM((NB,), jnp.int32),
            pltpu.VMEM((NB,), jnp.int32),
            pltpu.VMEM((PP,), jnp.int32),
            pltpu.VMEM((PP,), jnp.int32),
            pltpu.VMEM((PP, 128), jnp.int32),
            pltpu.VMEM((PP, 128), jnp.float32),
            pltpu.VMEM((PP, 128), jnp.float32),
            pltpu.SemaphoreType.DMA,
            pltpu.SemaphoreType.DMA,
            pltpu.SemaphoreType.DMA,
        ],
    )
    def k(row_hbm, col_hbm, flow_hbm, idx0_hbm, idx1_hbm, step0_hbm,
          step1_hbm, nlt_hbm, nst_hbm,
          rows0_out, rows1_out, cols0_out, cols1_out, ns0_out, ns1_out,
          idx_v, step_v, nlt_v, nst_v, ridx0, ridx1, fidx0, fidx1,
          cidx0, cidx1, flane_v, ns_v, fr_v, rows_v, cols_v,
          sem_rows, sem_flow, sem_cols):
        wid = lax.axis_index("s") * _NC + lax.axis_index("c")
        b0 = wid * NB
        pltpu.sync_copy(idx0_hbm.at[pl.ds(b0, NB)], idx_v.at[pl.ds(0, NB)])
        pltpu.sync_copy(idx1_hbm.at[pl.ds(b0, NB)], idx_v.at[pl.ds(NB, NB)])
        pltpu.sync_copy(step0_hbm.at[pl.ds(b0, NB)], step_v.at[pl.ds(0, NB)])
        pltpu.sync_copy(step1_hbm.at[pl.ds(b0, NB)], step_v.at[pl.ds(NB, NB)])
        pltpu.sync_copy(nlt_hbm, nlt_v)
        pltpu.sync_copy(nst_hbm, nst_v)
        nlt = nlt_v[...]
        nst = nst_v[...]
        iota = lax.broadcasted_iota(jnp.int32, (16,), 0)

        for c in range(NCH):
            a, ca = divmod(c, ACH)
            sl = pl.ds(c * _L, _L)
            iv = idx_v[sl]
            sv = step_v[sl]
            b = (b0 + ca * _L) + iota
            valid = lax.shift_right_logical(iv - nlt - 1, 31)
            lif = iv * valid
            tgt = ridx0 if a == 0 else ridx1
            tgt[pl.ds(ca * _L, _L)] = (b << 6) + lif
            nsp = sv + 1
            dummy = lax.shift_right_logical(nst - nsp, 31)
            dns = nsp * (1 - dummy)
            off = (lif << 5) + dns
            ftgt = fidx0 if a == 0 else fidx1
            ftgt[pl.ds(ca * _L, _L)] = (b << 4) + (off >> 7)
            flane_v[sl] = off & 127

        cp_f0 = pltpu.async_copy(flow_hbm.at[fidx0], fr_v.at[pl.ds(0, NB)], sem_flow)
        cp_f1 = pltpu.async_copy(flow_hbm.at[fidx1], fr_v.at[pl.ds(NB, NB)], sem_flow)
        cp_r0 = pltpu.async_copy(row_hbm.at[ridx0], rows_v.at[pl.ds(0, NB)], sem_rows)
        cp_r1 = pltpu.async_copy(row_hbm.at[ridx1], rows_v.at[pl.ds(NB, NB)], sem_rows)
        cp_f0.wait()
        cp_f1.wait()

        for c in range(NCH):
            a, ca = divmod(c, ACH)
            sl = pl.ds(c * _L, _L)
            fl = flane_v[sl]
            ns = jnp.zeros((_L,), jnp.int32)
            for j in range(_L):
                flj = fl[j]
                seg = fr_v[c * _L + j, pl.ds((flj >> 4) * _L, _L)]
                pick = take16(seg, jnp.full((_L,), flj & 15, jnp.int32))
                onehot = 1 - jnp.minimum(jnp.abs(iota - j), 1)
                ns = ns + pick * onehot
            ns_v[sl] = ns
            cs = jnp.minimum(jnp.maximum(ns - 1, 0), C - 1)
            b = (b0 + ca * _L) + iota
            tgt = cidx0 if a == 0 else cidx1
            tgt[pl.ds(ca * _L, _L)] = (b << 6) + cs

        cp_c0 = pltpu.async_copy(col_hbm.at[cidx0], cols_v.at[pl.ds(0, NB)], sem_cols)
        cp_c1 = pltpu.async_copy(col_hbm.at[cidx1], cols_v.at[pl.ds(NB, NB)], sem_cols)

        pltpu.sync_copy(ns_v.at[pl.ds(0, NB)], ns0_out.at[pl.ds(b0, NB)])
        pltpu.sync_copy(ns_v.at[pl.ds(NB, NB)], ns1_out.at[pl.ds(b0, NB)])
        cp_r0.wait()
        cp_r1.wait()
        pltpu.sync_copy(rows_v.at[pl.ds(0, NB)], rows0_out.at[pl.ds(b0, NB)])
        pltpu.sync_copy(rows_v.at[pl.ds(NB, NB)], rows1_out.at[pl.ds(b0, NB)])
        cp_c0.wait()
        cp_c1.wait()
        pltpu.sync_copy(cols_v.at[pl.ds(0, NB)], cols0_out.at[pl.ds(b0, NB)])
        pltpu.sync_copy(cols_v.at[pl.ds(NB, NB)], cols1_out.at[pl.ds(b0, NB)])

    return k(row_flat, col_flat, flow16, idx0, idx1, step0, step1,
             nlt_vec, nst_vec)


def _tc_combine_matmul(rows0, rows1, cols0, cols1, ns0, ns1,
                       idx0, idx1, step0, step1, nlt_s, nst_s, W, C):
    B, D = rows0.shape
    BM = 512

    def body(nlt_ref, nst_ref, r0_ref, r1_ref, c0_ref, c1_ref,
             ns0_ref, ns1_ref, i0_ref, i1_ref, s0_ref, s1_ref,
             w_ref, o_ref):
        nlt = nlt_ref[0]
        nst = nst_ref[0]
        w = w_ref[...]

        def arm(r_ref, c_ref, ns_ref, i_ref, s_ref, wslice):
            mr = (i_ref[...] <= nlt).astype(jnp.float32)[:, None]
            ns = ns_ref[...]
            dummy = s_ref[...] + 1 > nst
            mc = (jnp.logical_not(dummy) & (ns >= 1) & (ns <= C)
                  ).astype(jnp.float32)[:, None]
            e = r_ref[...] * mr + c_ref[...] * mc
            return lax.dot_general(e, wslice, (((1,), (1,)), ((), ())),
                                   preferred_element_type=jnp.float32)

        o_ref[...] = (arm(r0_ref, c0_ref, ns0_ref, i0_ref, s0_ref, w[:, :D])
                      + arm(r1_ref, c1_ref, ns1_ref, i1_ref, s1_ref, w[:, D:]))

    big = pl.BlockSpec((BM, D), lambda i, *_: (i, 0))
    vec = pl.BlockSpec((BM,), lambda i, *_: (i,))
    grid_spec = pltpu.PrefetchScalarGridSpec(
        num_scalar_prefetch=2,
        grid=(B // BM,),
        in_specs=[big, big, big, big, vec, vec, vec, vec, vec, vec,
                  pl.BlockSpec((D, 2 * D), lambda i, *_: (0, 0))],
        out_specs=pl.BlockSpec((BM, D), lambda i, *_: (i, 0)),
    )
    return pl.pallas_call(
        body,
        grid_spec=grid_spec,
        out_shape=jax.ShapeDtypeStruct((B, D), jnp.float32),
    )(nlt_s, nst_s, rows0, rows1, cols0, cols1, ns0, ns1,
      idx0, idx1, step0, step1, W)


def kernel(encoded_row, encoded_col, W, robot_lot_idx, robot_lot_step, flow,
           num_lot_type, num_step):
    B, R, D = encoded_row.shape
    C = encoded_col.shape[1]

    row_flat = encoded_row.reshape(B * R, D)
    col_flat = encoded_col.reshape(B * C, D)
    flow16 = flow.reshape(B * 16, 128).astype(jnp.int32)
    idx2 = robot_lot_idx.astype(jnp.int32)
    step2 = robot_lot_step.astype(jnp.int32)
    idx0, idx1 = idx2[:, 0], idx2[:, 1]
    step0, step1 = step2[:, 0], step2[:, 1]
    nlt_vec = jnp.full((16,), num_lot_type, jnp.int32)
    nst_vec = jnp.full((16,), num_step, jnp.int32)

    rows0, rows1, cols0, cols1, ns0, ns1 = _sc_gather(
        row_flat, col_flat, flow16, idx0, idx1, step0, step1,
        nlt_vec, nst_vec)

    nlt_s = jnp.full((1,), num_lot_type, jnp.int32)
    nst_s = jnp.full((1,), num_step, jnp.int32)
    return _tc_combine_matmul(rows0, rows1, cols0, cols1, ns0, ns1,
                              idx0, idx1, step0, step1, nlt_s, nst_s, W, C)
